# baseline (speedup 1.0000x reference)
import functools

import jax
import jax.numpy as jnp
from jax import lax
from jax.experimental import pallas as pl
from jax.experimental.pallas import tpu as pltpu
from jax.experimental.pallas import tpu_sc as plsc

D_MODEL = 256
EMBED_DIM = 64
NUM_HEADS = 4
HASH_RANGE = 65536
_B, _T = 4, 512
SEEDP1 = (1609.0, 5154.0, 6527.0, 2426.0)

NC, NS, LANES = 2, 16, 16
NW = NC * NS
NPOS = _B * _T
CHUNK = NPOS // NW
NCOMBO = 2 * NUM_HEADS
NROW = NCOMBO * CHUNK
NSEG = 4
SEG = NROW // NSEG


def _sc_body(tok_hbm, table_hbm, out_hbm, tok_v, idx_v, rows_v, acc_v, sem):
    wid = lax.axis_index("s") * NC + lax.axis_index("c")
    base = wid * CHUNK

    pltpu.sync_copy(tok_hbm.at[pl.ds(base, CHUNK + LANES)], tok_v)

    for j in range(CHUNK // LANES):
        t0 = tok_v[pl.ds(LANES * j, LANES)].astype(jnp.float32)
        t1 = tok_v[pl.ds(LANES * j + 1, LANES)].astype(jnp.float32)
        t2 = tok_v[pl.ds(LANES * j + 2, LANES)].astype(jnp.float32)
        for h in range(NUM_HEADS):
            s = SEEDP1[h]
            hv2 = t0 * s + t1 * s
            hv3 = hv2 + t2 * s
            i2 = (hv2.astype(jnp.int32) & (HASH_RANGE - 1)) + h * HASH_RANGE
            i3 = (hv3.astype(jnp.int32) & (HASH_RANGE - 1)) + h * HASH_RANGE
            f2 = h * CHUNK + LANES * j
            f3 = (NUM_HEADS + h) * CHUNK + LANES * j
            idx_v[f2 // SEG, pl.ds(f2 % SEG, LANES)] = i2
            idx_v[f3 // SEG, pl.ds(f3 % SEG, LANES)] = i3

    copies = [
        pltpu.async_copy(table_hbm.at[idx_v.at[k]], rows_v.at[k], sem)
        for k in range(NSEG)
    ]
    for c in copies:
        c.wait()

    @pl.when(base % _T == _T - CHUNK)
    def _():
        z = jnp.zeros((LANES,), jnp.float32)
        for p, combos in ((CHUNK - 2, range(NUM_HEADS, NCOMBO)),
                          (CHUNK - 1, range(NCOMBO))):
            for c in combos:
                g = c * CHUNK + p
                for dd in range(EMBED_DIM // LANES):
                    rows_v[g // SEG, g % SEG, pl.ds(LANES * dd, LANES)] = z

    for p in range(CHUNK):
        for dd in range(EMBED_DIM // LANES):
            acc = None
            for c in range(NCOMBO):
                g = c * CHUNK + p
                v = rows_v[g // SEG, g % SEG, pl.ds(LANES * dd, LANES)]
                acc = v if acc is None else acc + v
            acc_v[p, pl.ds(LANES * dd, LANES)] = acc * 0.25

    pltpu.sync_copy(acc_v, out_hbm.at[pl.ds(base, CHUNK)])


@functools.cache
def _sc_gather():
    return pl.kernel(
        _sc_body,
        out_type=jax.ShapeDtypeStruct((NPOS, EMBED_DIM), jnp.float32),
        mesh=plsc.VectorSubcoreMesh(core_axis_name="c", subcore_axis_name="s",
                                    num_cores=NC, num_subcores=NS),
        compiler_params=pltpu.CompilerParams(use_tc_tiling_on_sc=False),
        scratch_types=[
            pltpu.VMEM((CHUNK + LANES,), jnp.int32),
            pltpu.VMEM((NSEG, SEG), jnp.int32),
            pltpu.VMEM((NSEG, SEG, EMBED_DIM), jnp.float32),
            pltpu.VMEM((CHUNK, EMBED_DIM), jnp.float32),
            pltpu.SemaphoreType.DMA,
        ],
    )


def _dense_body(hid_ref, seq_ref, wh_ref, bh_ref, wg1_ref, bg1_ref,
                wg2_ref, bg2_ref, out_ref):
    hid = hid_ref[...]
    seq = seq_ref[...]
    proj = lax.dot_general(seq, wh_ref[...], (((1,), (1,)), ((), ())),
                           preferred_element_type=jnp.float32) + bh_ref[...]
    h = hid + proj
    g1 = lax.dot_general(h, wg1_ref[...], (((1,), (1,)), ((), ())),
                         preferred_element_type=jnp.float32) + bg1_ref[...]
    g1 = 0.5 * g1 * (1.0 + lax.erf(g1 * (2.0 ** -0.5)))
    g2w = jnp.broadcast_to(wg2_ref[...], (D_MODEL, D_MODEL // 2))
    g2 = lax.dot_general(g1, g2w, (((1,), (1,)), ((), ())),
                         preferred_element_type=jnp.float32) + bg2_ref[0, 0]
    gate = jax.nn.sigmoid(g2)
    out_ref[...] = hid + gate * proj


def _dense(hid2d, seq, W_hid, b_hid, W_g1, b_g1, W_g2, b_g2):
    return pl.pallas_call(
        _dense_body,
        out_shape=jax.ShapeDtypeStruct((NPOS, D_MODEL), jnp.float32),
    )(hid2d, seq, W_hid, b_hid.reshape(1, -1), W_g1, b_g1.reshape(1, -1),
      W_g2, b_g2.reshape(1, 1))


def kernel(token_ids, hidden_state, embeddings, W_hid, b_hid, W_g1, b_g1,
           W_g2, b_g2):
    tok = token_ids.reshape(-1)
    tok_pad = jnp.concatenate([tok, jnp.zeros((LANES,), jnp.int32)])
    table = embeddings.reshape(NUM_HEADS * HASH_RANGE, EMBED_DIM)
    seq = _sc_gather()(tok_pad, table)
    hid2d = hidden_state.reshape(NPOS, D_MODEL)
    out = _dense(hid2d, seq, W_hid, b_hid, W_g1, b_g1, W_g2, b_g2)
    return out.reshape(hidden_state.shape)
